# Initial kernel scaffold; baseline (speedup 1.0000x reference)
import jax
import jax.numpy as jnp
from jax.experimental import pallas as pl


def kernel(features, batch, W1, b1, gamma, beta, W2, b2):
    raise NotImplementedError("write your pallas kernel here")


# speedup vs baseline: 5.7864x; 5.7864x over previous
import functools

import jax
import jax.numpy as jnp
from jax import lax
from jax.experimental import pallas as pl
from jax.experimental.pallas import tpu as pltpu
from jax.experimental.pallas import tpu_sc as plsc

NUM_SEG = 512
DIM = 128
HID = DIM // 2
ROWS = 320000
BLK = 128
NBLK = ROWS // BLK
NC, NS = 2, 16
NW = NC * NS
BASE_BLKS = NBLK // NW
EXTRA = NBLK - BASE_BLKS * NW


def _pool_body(feat, ids, sums, cnts, rows_v, idx_v, ones_v, zrow_v, acc_s, cnt_s):
    cid = lax.axis_index("c")
    sid = lax.axis_index("s")
    wid = sid * NC + cid

    for i in range(NUM_SEG // NS):
        for j in range(DIM // 16):
            zrow_v[i, pl.ds(j * 16, 16)] = jnp.zeros((16,), jnp.float32)
    for j in range(BLK // 16):
        ones_v[pl.ds(j * 16, 16)] = jnp.ones((16,), jnp.float32)
    pltpu.sync_copy(zrow_v, acc_s.at[pl.ds(sid * (NUM_SEG // NS), NUM_SEG // NS)])
    pltpu.sync_copy(zrow_v.at[0, pl.ds(0, NUM_SEG // NS)],
                    cnt_s.at[pl.ds(sid * (NUM_SEG // NS), NUM_SEG // NS)])
    plsc.subcore_barrier()

    def do_block(g):
        r0 = g * BLK
        pltpu.sync_copy(ids.at[pl.ds(r0, BLK)], idx_v.at[0])
        pltpu.sync_copy(feat.at[pl.ds(r0, BLK)], rows_v)
        pltpu.sync_copy(rows_v, acc_s.at[idx_v.at[0]], add=True)
        pltpu.sync_copy(ones_v, cnt_s.at[idx_v.at[0]], add=True)

    base = wid * BASE_BLKS

    def body(i, carry):
        do_block(base + i)
        return carry

    lax.fori_loop(0, BASE_BLKS, body, 0)

    @pl.when(wid < EXTRA)
    def _():
        do_block(NW * BASE_BLKS + wid)

    plsc.subcore_barrier()

    @pl.when(sid == 0)
    def _():
        pltpu.sync_copy(acc_s, sums.at[cid])
        pltpu.sync_copy(cnt_s, cnts.at[cid])


_pool = functools.partial(
    pl.kernel,
    out_type=[
        jax.ShapeDtypeStruct((NC, NUM_SEG, DIM), jnp.float32),
        jax.ShapeDtypeStruct((NC, NUM_SEG), jnp.float32),
    ],
    mesh=plsc.VectorSubcoreMesh(core_axis_name="c", subcore_axis_name="s"),
    scratch_types=[
        pltpu.VMEM((BLK, DIM), jnp.float32),
        pltpu.VMEM((1, BLK), jnp.int32),
        pltpu.VMEM((BLK,), jnp.float32),
        pltpu.VMEM((NUM_SEG // NS, DIM), jnp.float32),
        pltpu.VMEM_SHARED((NUM_SEG, DIM), jnp.float32),
        pltpu.VMEM_SHARED((NUM_SEG,), jnp.float32),
    ],
)(_pool_body)


def _head_body(sums, cnts, W1, b1, gamma, beta, W2, b2, out):
    s = sums[0] + sums[1]
    c = cnts[0] + cnts[1]
    pooled = s / jnp.maximum(c, 1.0)
    h = lax.dot_general(pooled, W1[...], (((1,), (1,)), ((), ())),
                        preferred_element_type=jnp.float32)
    h = h + b1[...]
    mean = jnp.mean(h, axis=1, keepdims=True)
    var = jnp.mean((h - mean) * (h - mean), axis=1, keepdims=True)
    h = gamma[...] * (h - mean) * lax.rsqrt(var + 1e-5) + beta[...]
    h = jnp.where(h >= 0, h, 0.01 * h)
    out[...] = jnp.sum(h * W2[...], axis=1, keepdims=True) + b2[...]


def _head(sums, cnts, W1, b1, gamma, beta, W2, b2):
    return pl.pallas_call(
        _head_body,
        out_shape=jax.ShapeDtypeStruct((NUM_SEG, 1), jnp.float32),
    )(sums, cnts, W1, b1, gamma, beta, W2, b2)


def kernel(features, batch, W1, b1, gamma, beta, W2, b2):
    ids = batch.astype(jnp.int32)
    sums, cnts = _pool(features, ids)
    return _head(sums, cnts.reshape(NC, NUM_SEG, 1), W1,
                 b1.reshape(1, HID), gamma.reshape(1, HID),
                 beta.reshape(1, HID), W2, b2.reshape(1, 1))


# speedup vs baseline: 8.5617x; 1.4796x over previous
import functools

import jax
import jax.numpy as jnp
from jax import lax
from jax.experimental import pallas as pl
from jax.experimental.pallas import tpu as pltpu
from jax.experimental.pallas import tpu_sc as plsc

NUM_SEG = 512
DIM = 128
HID = DIM // 2
ROWS = 320000
BLK = 128
NBLK = ROWS // BLK
NC, NS = 2, 16
NW = NC * NS
BASE_BLKS = NBLK // NW
EXTRA = NBLK - BASE_BLKS * NW


NBUF = 3
NSTEP = BASE_BLKS // NBUF


def _pool_body(feat, ids, sums, cnts, rows_v, idx_v, ones_v, zrow_v,
               acc_s, cnt_s, in_sems, sc_sems):
    cid = lax.axis_index("c")
    sid = lax.axis_index("s")
    wid = sid * NC + cid

    for i in range(NUM_SEG // NS):
        for j in range(DIM // 16):
            zrow_v[i, pl.ds(j * 16, 16)] = jnp.zeros((16,), jnp.float32)
    for j in range(BLK // 16):
        ones_v[pl.ds(j * 16, 16)] = jnp.ones((16,), jnp.float32)
    pltpu.sync_copy(zrow_v, acc_s.at[pl.ds(sid * (NUM_SEG // NS), NUM_SEG // NS)])
    pltpu.sync_copy(zrow_v.at[0, pl.ds(0, NUM_SEG // NS)],
                    cnt_s.at[pl.ds(sid * (NUM_SEG // NS), NUM_SEG // NS)])
    plsc.subcore_barrier()

    def fire_in(b, g):
        r0 = g * BLK
        pltpu.async_copy(ids.at[pl.ds(r0, BLK)], idx_v.at[b], in_sems.at[b])
        pltpu.async_copy(feat.at[pl.ds(r0, BLK)], rows_v.at[b], in_sems.at[b])

    def wait_in(b):
        pltpu.make_async_copy(ids.at[pl.ds(0, BLK)], idx_v.at[b],
                              in_sems.at[b]).wait()
        pltpu.make_async_copy(feat.at[pl.ds(0, BLK)], rows_v.at[b],
                              in_sems.at[b]).wait()

    def fire_sc(b):
        pltpu.async_copy(rows_v.at[b], acc_s.at[idx_v.at[b]], sc_sems.at[b],
                         add=True)
        pltpu.async_copy(ones_v, cnt_s.at[idx_v.at[b]], sc_sems.at[b],
                         add=True)

    def wait_sc(b):
        pltpu.make_async_copy(rows_v.at[b], acc_s.at[idx_v.at[b]],
                              sc_sems.at[b]).wait()
        pltpu.make_async_copy(ones_v, cnt_s.at[idx_v.at[b]],
                              sc_sems.at[b]).wait()

    base = wid * BASE_BLKS
    for b in range(NBUF):
        fire_in(b, base + b)

    def outer(j, carry):
        for b in range(NBUF):
            wait_in(b)
            fire_sc(b)
            wait_sc(b)

            @pl.when(j < NSTEP - 1)
            def _():
                fire_in(b, base + NBUF * (j + 1) + b)
        return carry

    lax.fori_loop(0, NSTEP, outer, 0)

    @pl.when(wid < EXTRA)
    def _():
        fire_in(0, NW * BASE_BLKS + wid)
        wait_in(0)
        fire_sc(0)
        wait_sc(0)

    plsc.subcore_barrier()

    @pl.when(sid == 0)
    def _():
        pltpu.sync_copy(acc_s, sums.at[cid])
        pltpu.sync_copy(cnt_s, cnts.at[cid])


_pool = functools.partial(
    pl.kernel,
    out_type=[
        jax.ShapeDtypeStruct((NC, NUM_SEG, DIM), jnp.float32),
        jax.ShapeDtypeStruct((NC, NUM_SEG), jnp.float32),
    ],
    mesh=plsc.VectorSubcoreMesh(core_axis_name="c", subcore_axis_name="s"),
    scratch_types=[
        pltpu.VMEM((NBUF, BLK, DIM), jnp.float32),
        pltpu.VMEM((NBUF, BLK), jnp.int32),
        pltpu.VMEM((BLK,), jnp.float32),
        pltpu.VMEM((NUM_SEG // NS, DIM), jnp.float32),
        pltpu.VMEM_SHARED((NUM_SEG, DIM), jnp.float32),
        pltpu.VMEM_SHARED((NUM_SEG,), jnp.float32),
        pltpu.SemaphoreType.DMA((NBUF,)),
        pltpu.SemaphoreType.DMA((NBUF,)),
    ],
)(_pool_body)


def _head_body(sums, cnts, W1, b1, gamma, beta, W2, b2, out):
    s = sums[0] + sums[1]
    c = cnts[0] + cnts[1]
    pooled = s / jnp.maximum(c, 1.0)
    h = lax.dot_general(pooled, W1[...], (((1,), (1,)), ((), ())),
                        preferred_element_type=jnp.float32)
    h = h + b1[...]
    mean = jnp.mean(h, axis=1, keepdims=True)
    var = jnp.mean((h - mean) * (h - mean), axis=1, keepdims=True)
    h = gamma[...] * (h - mean) * lax.rsqrt(var + 1e-5) + beta[...]
    h = jnp.where(h >= 0, h, 0.01 * h)
    out[...] = jnp.sum(h * W2[...], axis=1, keepdims=True) + b2[...]


def _head(sums, cnts, W1, b1, gamma, beta, W2, b2):
    return pl.pallas_call(
        _head_body,
        out_shape=jax.ShapeDtypeStruct((NUM_SEG, 1), jnp.float32),
    )(sums, cnts, W1, b1, gamma, beta, W2, b2)


def kernel(features, batch, W1, b1, gamma, beta, W2, b2):
    ids = batch.astype(jnp.int32)
    sums, cnts = _pool(features, ids)
    return _head(sums, cnts.reshape(NC, NUM_SEG, 1), W1,
                 b1.reshape(1, HID), gamma.reshape(1, HID),
                 beta.reshape(1, HID), W2, b2.reshape(1, 1))
